# baseline (speedup 1.0000x reference)
import jax
import jax.numpy as jnp
from jax.experimental import pallas as pl
from jax.experimental.pallas import tpu as pltpu

_BI = 512
_BK = 512
_PREC = jax.lax.Precision.DEFAULT


def _pass1(adj_ref, seq_ref, w_ref, bias_ref, out_ref, sf_ref, acc_ref):
    i = pl.program_id(0)
    k = pl.program_id(1)
    nk = pl.num_programs(1)

    @pl.when(i == 0)
    def _compute_sf_block():
        sf_ref[pl.ds(k * _BK, _BK), :] = jax.lax.dot_general(
            seq_ref[pl.ds(k * _BK, _BK), :],
            w_ref[...],
            (((1,), (1,)), ((), ())),
            preferred_element_type=jnp.float32,
            precision=_PREC,
        )

    @pl.when(k == 0)
    def _init():
        acc_ref[...] = jnp.zeros_like(acc_ref)

    acc_ref[...] += jax.lax.dot_general(
        adj_ref[...],
        sf_ref[pl.ds(k * _BK, _BK), :],
        (((1,), (0,)), ((), ())),
        preferred_element_type=jnp.float32,
        precision=_PREC,
    )

    @pl.when(k == nk - 1)
    def _epilogue():
        out_ref[...] = acc_ref[...] + bias_ref[...]


def _pass2(adj_ref, h_ref, bias_ref, a_ref, out_ref, acc_ref):
    k = pl.program_id(1)
    nk = pl.num_programs(1)

    @pl.when(k == 0)
    def _init():
        acc_ref[...] = jnp.zeros_like(acc_ref)

    acc_ref[...] += jax.lax.dot_general(
        adj_ref[...],
        h_ref[pl.ds(k * _BK, _BK), :],
        (((1,), (0,)), ((), ())),
        preferred_element_type=jnp.float32,
        precision=_PREC,
    )

    @pl.when(k == nk - 1)
    def _epilogue():
        o = acc_ref[...] + bias_ref[...]
        out_ref[...] = jnp.where(o > 0, o, a_ref[0, 0] * o)


def kernel(seq, adj, du, W, bias, prelu_a):
    del du
    (b, n, f_in) = seq.shape
    f_out = W.shape[0]
    seq2 = seq.reshape(n, f_in)
    adj2 = adj.reshape(n, n)
    bias2 = bias.reshape(1, f_out)
    a2 = jnp.reshape(prelu_a, (1, 1)).astype(jnp.float32)

    ni = n // _BI
    nk = n // _BK

    h = pl.pallas_call(
        _pass1,
        grid=(ni, nk),
        in_specs=[
            pl.BlockSpec((_BI, _BK), lambda i, k: (i, k)),
            pl.BlockSpec((n, f_in), lambda i, k: (0, 0)),
            pl.BlockSpec((f_out, f_in), lambda i, k: (0, 0)),
            pl.BlockSpec((1, f_out), lambda i, k: (0, 0)),
        ],
        out_specs=pl.BlockSpec((_BI, f_out), lambda i, k: (i, 0)),
        out_shape=jax.ShapeDtypeStruct((n, f_out), jnp.float32),
        scratch_shapes=[
            pltpu.VMEM((n, f_out), jnp.float32),
            pltpu.VMEM((_BI, f_out), jnp.float32),
        ],
    )(adj2, seq2, W, bias2)

    out = pl.pallas_call(
        _pass2,
        grid=(ni, nk),
        in_specs=[
            pl.BlockSpec((_BI, _BK), lambda i, k: (i, k)),
            pl.BlockSpec((n, f_out), lambda i, k: (0, 0)),
            pl.BlockSpec((1, f_out), lambda i, k: (0, 0)),
            pl.BlockSpec((1, 1), lambda i, k: (0, 0)),
        ],
        out_specs=pl.BlockSpec((_BI, f_out), lambda i, k: (i, 0)),
        out_shape=jax.ShapeDtypeStruct((n, f_out), jnp.float32),
        scratch_shapes=[
            pltpu.VMEM((_BI, f_out), jnp.float32),
        ],
    )(adj2, h, bias2, a2)

    return out.reshape(b, n, f_out)


# speedup vs baseline: 2.6192x; 2.6192x over previous
import jax
import jax.numpy as jnp
from jax.experimental import pallas as pl
from jax.experimental.pallas import tpu as pltpu

_BI = 256


def _fused(adj_ref, seq_ref, w_ref, bias_ref, a_ref, out_ref,
           adjbf_ref, sf_ref, h_ref):
    p = pl.program_id(0)
    i = pl.program_id(1)
    rows = pl.ds(i * _BI, _BI)

    @pl.when(jnp.logical_and(p == 0, i == 0))
    def _compute_sf():
        sf_ref[...] = jax.lax.dot_general(
            seq_ref[...].astype(jnp.bfloat16),
            w_ref[...].astype(jnp.bfloat16),
            (((1,), (1,)), ((), ())),
            preferred_element_type=jnp.float32,
        ).astype(jnp.bfloat16)

    @pl.when(p == 0)
    def _phase0():
        blk = adj_ref[...].astype(jnp.bfloat16)
        adjbf_ref[rows, :] = blk
        h = jax.lax.dot_general(
            blk, sf_ref[...],
            (((1,), (0,)), ((), ())),
            preferred_element_type=jnp.float32,
        ) + bias_ref[...]
        h_ref[rows, :] = h.astype(jnp.bfloat16)

    @pl.when(p == 1)
    def _phase1():
        o = jax.lax.dot_general(
            adjbf_ref[rows, :], h_ref[...],
            (((1,), (0,)), ((), ())),
            preferred_element_type=jnp.float32,
        ) + bias_ref[...]
        out_ref[...] = jnp.where(o > 0, o, a_ref[0, 0] * o)


def kernel(seq, adj, du, W, bias, prelu_a):
    del du
    (b, n, f_in) = seq.shape
    f_out = W.shape[0]
    seq2 = seq.reshape(n, f_in)
    adj2 = adj.reshape(n, n)
    bias2 = bias.reshape(1, f_out)
    a2 = jnp.reshape(prelu_a, (1, 1)).astype(jnp.float32)

    ni = n // _BI

    out = pl.pallas_call(
        _fused,
        grid=(2, ni),
        in_specs=[
            pl.BlockSpec((_BI, n), lambda p, i: ((1 - p) * i + p * (ni - 1), 0)),
            pl.BlockSpec((n, f_in), lambda p, i: (0, 0)),
            pl.BlockSpec((f_out, f_in), lambda p, i: (0, 0)),
            pl.BlockSpec((1, f_out), lambda p, i: (0, 0)),
            pl.BlockSpec((1, 1), lambda p, i: (0, 0)),
        ],
        out_specs=pl.BlockSpec((_BI, f_out), lambda p, i: (i, 0)),
        out_shape=jax.ShapeDtypeStruct((n, f_out), jnp.float32),
        scratch_shapes=[
            pltpu.VMEM((n, n), jnp.bfloat16),
            pltpu.VMEM((n, f_out), jnp.bfloat16),
            pltpu.VMEM((n, f_out), jnp.bfloat16),
        ],
        compiler_params=pltpu.CompilerParams(
            vmem_limit_bytes=64 * 1024 * 1024,
        ),
    )(adj2, seq2, W, bias2, a2)

    return out.reshape(b, n, f_out)


# speedup vs baseline: 2.6499x; 1.0117x over previous
import jax
import jax.numpy as jnp
from jax.experimental import pallas as pl
from jax.experimental.pallas import tpu as pltpu

_BI = 256


def _fused(adj_ref, seq_ref, w_ref, bias_ref, a_ref, out_ref,
           adjbf_ref, sf_ref, h_ref):
    g = pl.program_id(0)
    ni = (pl.num_programs(0) - 1) // 2

    @pl.when(g == 0)
    def _compute_sf():
        sf_ref[...] = jax.lax.dot_general(
            seq_ref[...].astype(jnp.bfloat16),
            w_ref[...].astype(jnp.bfloat16),
            (((1,), (1,)), ((), ())),
            preferred_element_type=jnp.float32,
        ).astype(jnp.bfloat16)

    @pl.when(g < ni)
    def _cast_block():
        adjbf_ref[pl.ds(g * _BI, _BI), :] = adj_ref[...].astype(jnp.bfloat16)

    @pl.when(jnp.logical_and(g > 0, g <= ni))
    def _h_block():
        rows = pl.ds((g - 1) * _BI, _BI)
        h = jax.lax.dot_general(
            adjbf_ref[rows, :], sf_ref[...],
            (((1,), (0,)), ((), ())),
            preferred_element_type=jnp.float32,
        ) + bias_ref[...]
        h_ref[rows, :] = h.astype(jnp.bfloat16)

    @pl.when(g > ni)
    def _out_block():
        rows = pl.ds((g - ni - 1) * _BI, _BI)
        o = jax.lax.dot_general(
            adjbf_ref[rows, :], h_ref[...],
            (((1,), (0,)), ((), ())),
            preferred_element_type=jnp.float32,
        ) + bias_ref[...]
        out_ref[...] = jnp.where(o > 0, o, a_ref[0, 0] * o)


def kernel(seq, adj, du, W, bias, prelu_a):
    del du
    (b, n, f_in) = seq.shape
    f_out = W.shape[0]
    seq2 = seq.reshape(n, f_in)
    adj2 = adj.reshape(n, n)
    bias2 = bias.reshape(1, f_out)
    a2 = jnp.reshape(prelu_a, (1, 1)).astype(jnp.float32)

    ni = n // _BI

    out = pl.pallas_call(
        _fused,
        grid=(2 * ni + 1,),
        in_specs=[
            pl.BlockSpec((_BI, n), lambda g: (jnp.minimum(g, ni - 1), 0)),
            pl.BlockSpec((n, f_in), lambda g: (0, 0)),
            pl.BlockSpec((f_out, f_in), lambda g: (0, 0)),
            pl.BlockSpec((1, f_out), lambda g: (0, 0)),
            pl.BlockSpec((1, 1), lambda g: (0, 0)),
        ],
        out_specs=pl.BlockSpec(
            (_BI, f_out), lambda g: (jnp.maximum(g - ni - 1, 0), 0)),
        out_shape=jax.ShapeDtypeStruct((n, f_out), jnp.float32),
        scratch_shapes=[
            pltpu.VMEM((n, n), jnp.bfloat16),
            pltpu.VMEM((n, f_out), jnp.bfloat16),
            pltpu.VMEM((n, f_out), jnp.bfloat16),
        ],
        compiler_params=pltpu.CompilerParams(
            vmem_limit_bytes=64 * 1024 * 1024,
        ),
    )(adj2, seq2, W, bias2, a2)

    return out.reshape(b, n, f_out)


# speedup vs baseline: 3.0384x; 1.1466x over previous
import jax
import jax.numpy as jnp
from jax.experimental import pallas as pl
from jax.experimental.pallas import tpu as pltpu

_BI = 512


def _fused(adj_ref, seq_ref, w_ref, bias_ref, a_ref, out_ref,
           adjbf_ref, sf_ref, h_ref):
    p = pl.program_id(0)
    i = pl.program_id(1)
    rows = pl.ds(i * _BI, _BI)

    @pl.when(jnp.logical_and(p == 0, i == 0))
    def _compute_sf():
        sf_ref[...] = jax.lax.dot_general(
            seq_ref[...], w_ref[...],
            (((1,), (1,)), ((), ())),
            preferred_element_type=jnp.float32,
        )

    @pl.when(p == 0)
    def _phase0():
        blk = adj_ref[...]
        adjbf_ref[rows, :] = blk.astype(jnp.bfloat16)
        h = jax.lax.dot_general(
            blk, sf_ref[...],
            (((1,), (0,)), ((), ())),
            preferred_element_type=jnp.float32,
        ) + bias_ref[...]
        h_ref[rows, :] = h.astype(jnp.bfloat16)

    @pl.when(p == 1)
    def _phase1():
        o = jax.lax.dot_general(
            adjbf_ref[rows, :], h_ref[...],
            (((1,), (0,)), ((), ())),
            preferred_element_type=jnp.float32,
        ) + bias_ref[...]
        out_ref[...] = jnp.where(o > 0, o, a_ref[0, 0] * o)


def kernel(seq, adj, du, W, bias, prelu_a):
    del du
    (b, n, f_in) = seq.shape
    f_out = W.shape[0]
    seq2 = seq.reshape(n, f_in)
    adj2 = adj.reshape(n, n)
    bias2 = bias.reshape(1, f_out)
    a2 = jnp.reshape(prelu_a, (1, 1)).astype(jnp.float32)

    ni = n // _BI

    out = pl.pallas_call(
        _fused,
        grid=(2, ni),
        in_specs=[
            pl.BlockSpec((_BI, n), lambda p, i: ((1 - p) * i + p * (ni - 1), 0)),
            pl.BlockSpec((n, f_in), lambda p, i: (0, 0)),
            pl.BlockSpec((f_out, f_in), lambda p, i: (0, 0)),
            pl.BlockSpec((1, f_out), lambda p, i: (0, 0)),
            pl.BlockSpec((1, 1), lambda p, i: (0, 0)),
        ],
        out_specs=pl.BlockSpec((_BI, f_out), lambda p, i: (i, 0)),
        out_shape=jax.ShapeDtypeStruct((n, f_out), jnp.float32),
        scratch_shapes=[
            pltpu.VMEM((n, n), jnp.bfloat16),
            pltpu.VMEM((n, f_out), jnp.float32),
            pltpu.VMEM((n, f_out), jnp.bfloat16),
        ],
        compiler_params=pltpu.CompilerParams(
            vmem_limit_bytes=64 * 1024 * 1024,
        ),
    )(adj2, seq2, W, bias2, a2)

    return out.reshape(b, n, f_out)


# speedup vs baseline: 3.1634x; 1.0412x over previous
import jax
import jax.numpy as jnp
from jax.experimental import pallas as pl
from jax.experimental.pallas import tpu as pltpu

_BI = 512
_BO = 1024


def _fused(adj_ref, seq_ref, w_ref, bias_ref, a_ref, out_ref,
           adjbf_ref, sf_ref, h_ref):
    g = pl.program_id(0)
    n = adjbf_ref.shape[0]
    ni = n // _BI

    @pl.when(g == 0)
    def _compute_sf():
        sf_ref[...] = jax.lax.dot_general(
            seq_ref[...], w_ref[...],
            (((1,), (1,)), ((), ())),
            preferred_element_type=jnp.float32,
        )

    @pl.when(g < ni)
    def _phase0():
        rows = pl.ds(g * _BI, _BI)
        blk = adj_ref[...]
        adjbf_ref[rows, :] = blk.astype(jnp.bfloat16)
        h = jax.lax.dot_general(
            blk, sf_ref[...],
            (((1,), (0,)), ((), ())),
            preferred_element_type=jnp.float32,
        ) + bias_ref[...]
        h_ref[rows, :] = h.astype(jnp.bfloat16)

    @pl.when(g >= ni)
    def _phase1():
        rows = pl.ds((g - ni) * _BO, _BO)
        o = jax.lax.dot_general(
            adjbf_ref[rows, :], h_ref[...],
            (((1,), (0,)), ((), ())),
            preferred_element_type=jnp.float32,
        ) + bias_ref[...]
        out_ref[...] = jnp.where(o > 0, o, a_ref[0, 0] * o)


def kernel(seq, adj, du, W, bias, prelu_a):
    del du
    (b, n, f_in) = seq.shape
    f_out = W.shape[0]
    seq2 = seq.reshape(n, f_in)
    adj2 = adj.reshape(n, n)
    bias2 = bias.reshape(1, f_out)
    a2 = jnp.reshape(prelu_a, (1, 1)).astype(jnp.float32)

    ni = n // _BI
    no = n // _BO

    out = pl.pallas_call(
        _fused,
        grid=(ni + no,),
        in_specs=[
            pl.BlockSpec((_BI, n), lambda g: (jnp.minimum(g, ni - 1), 0)),
            pl.BlockSpec((n, f_in), lambda g: (0, 0)),
            pl.BlockSpec((f_out, f_in), lambda g: (0, 0)),
            pl.BlockSpec((1, f_out), lambda g: (0, 0)),
            pl.BlockSpec((1, 1), lambda g: (0, 0)),
        ],
        out_specs=pl.BlockSpec(
            (_BO, f_out), lambda g: (jnp.maximum(g - ni, 0), 0)),
        out_shape=jax.ShapeDtypeStruct((n, f_out), jnp.float32),
        scratch_shapes=[
            pltpu.VMEM((n, n), jnp.bfloat16),
            pltpu.VMEM((n, f_out), jnp.float32),
            pltpu.VMEM((n, f_out), jnp.bfloat16),
        ],
        compiler_params=pltpu.CompilerParams(
            vmem_limit_bytes=64 * 1024 * 1024,
        ),
    )(adj2, seq2, W, bias2, a2)

    return out.reshape(b, n, f_out)
